# baseline (speedup 1.0000x reference)
import functools

import jax
import jax.numpy as jnp
from jax import lax
from jax.experimental import pallas as pl
from jax.experimental.pallas import tpu as pltpu
from jax.experimental.pallas import tpu_sc as plsc

B, P, D, OUT = 4096, 32, 128, 128
LANES = 16
NC, NS = 2, 16
NW = NC * NS
CB = 12
DBLK = D // LANES

B_SC = 1536
B_TC = B - B_SC
EPW = B_SC // NW
NCHUNKS = EPW // CB
TC_BLK = 512


def _sc_segment_sum(x_flat):
  mesh = plsc.VectorSubcoreMesh(core_axis_name="c", subcore_axis_name="s")

  @functools.partial(
      pl.kernel,
      out_type=jax.ShapeDtypeStruct((B_SC, D), jnp.float32),
      mesh=mesh,
      scratch_types=[
          pltpu.VMEM((CB * P, D), jnp.float32),
          pltpu.VMEM((CB * P, D), jnp.float32),
          pltpu.VMEM((EPW, D), jnp.float32),
          pltpu.SemaphoreType.DMA,
          pltpu.SemaphoreType.DMA,
      ],
  )
  def k(x_hbm, out_hbm, buf0, buf1, stage, sem0, sem1):
    wid = lax.axis_index("s") * NC + lax.axis_index("c")
    elem0 = wid * EPW
    bufs = (buf0, buf1)
    sems = (sem0, sem1)

    def fetch(c, par):
      pltpu.async_copy(
          x_hbm.at[pl.ds((elem0 + c * CB) * P, CB * P)], bufs[par], sems[par]
      )

    def wait(c, par):
      pltpu.make_async_copy(
          x_hbm.at[pl.ds((elem0 + c * CB) * P, CB * P)], bufs[par], sems[par]
      ).wait()

    def reduce_chunk(c, par):
      buf = bufs[par]

      def elem_body(e, carry):
        base = e * P
        dsls = [pl.ds(d * LANES, LANES) for d in range(DBLK)]
        row_out = c * CB + e
        for d0 in range(0, DBLK, 4):
          grp = dsls[d0:d0 + 4]
          accs = [buf[base, dsl] for dsl in grp]
          for p in range(1, P):
            for i, dsl in enumerate(grp):
              accs[i] = accs[i] + buf[base + p, dsl]
          for i, dsl in enumerate(grp):
            stage[row_out, dsl] = accs[i]
        return carry

      lax.fori_loop(0, CB, elem_body, 0)

    fetch(0, 0)
    fetch(1, 1)

    @pl.loop(0, NCHUNKS - (NCHUNKS % 2), step=2)
    def _ring(g):
      wait(g, 0)
      reduce_chunk(g, 0)
      pl.when(g + 2 < NCHUNKS)(lambda: fetch(g + 2, 0))
      wait(g + 1, 1)
      reduce_chunk(g + 1, 1)
      pl.when(g + 3 < NCHUNKS)(lambda: fetch(g + 3, 1))

    if NCHUNKS % 2:
      wait(NCHUNKS - 1, 0)
      reduce_chunk(NCHUNKS - 1, 0)

    pltpu.sync_copy(stage, out_hbm.at[pl.ds(elem0, EPW)])

  return k(x_flat)


def _mm_bias(s, w_ref, b_ref):
  y = lax.dot_general(
      s, w_ref[...], (((1,), (1,)), ((), ())),
      preferred_element_type=jnp.float32,
  )
  return y + jnp.float32(P) * b_ref[...]


def _tc_fused(x_full, w, b2d):
  def body(x_ref, w_ref, b_ref, o_ref):
    o_ref[...] = _mm_bias(jnp.sum(x_ref[...], axis=1), w_ref, b_ref)

  off = B_SC // TC_BLK
  return pl.pallas_call(
      body,
      grid=(B_TC // TC_BLK,),
      in_specs=[
          pl.BlockSpec((TC_BLK, P, D), lambda i: (i + off, 0, 0)),
          pl.BlockSpec((OUT, D), lambda i: (0, 0)),
          pl.BlockSpec((1, OUT), lambda i: (0, 0)),
      ],
      out_specs=pl.BlockSpec((TC_BLK, OUT), lambda i: (i + off, 0)),
      out_shape=jax.ShapeDtypeStruct((B, OUT), jnp.float32),
  )(x_full, w, b2d)


def _tc_linear(s, w, b2d, out_partial):
  def mm(s_ref, w_ref, b_ref, _, o_ref):
    o_ref[...] = _mm_bias(s_ref[...], w_ref, b_ref)

  return pl.pallas_call(
      mm,
      grid=(1,),
      in_specs=[
          pl.BlockSpec((B_SC, D), lambda i: (0, 0)),
          pl.BlockSpec((OUT, D), lambda i: (0, 0)),
          pl.BlockSpec((1, OUT), lambda i: (0, 0)),
          pl.BlockSpec(memory_space=pl.ANY),
      ],
      out_specs=pl.BlockSpec((B_SC, OUT), lambda i: (0, 0)),
      out_shape=jax.ShapeDtypeStruct((B, OUT), jnp.float32),
      input_output_aliases={3: 0},
  )(s, w, b2d, out_partial)


@jax.jit
def kernel(pert_batch, W, b):
  b2d = b.reshape(1, OUT)
  x_flat = pert_batch.reshape(B * P, D)
  s_sc = _sc_segment_sum(x_flat)
  out_partial = _tc_fused(pert_batch, W, b2d)
  return _tc_linear(s_sc, W, b2d, out_partial)


# speedup vs baseline: 1.0086x; 1.0086x over previous
import functools

import jax
import jax.numpy as jnp
from jax import lax
from jax.experimental import pallas as pl
from jax.experimental.pallas import tpu as pltpu
from jax.experimental.pallas import tpu_sc as plsc

B, P, D, OUT = 4096, 32, 128, 128
LANES = 16
NC, NS = 2, 16
NW = NC * NS
CB = 8
DBLK = D // LANES

B_SC = 1536
B_TC = B - B_SC
EPW = B_SC // NW
NCHUNKS = EPW // CB
TC_BLK = 512


def _sc_segment_sum(x_flat):
  mesh = plsc.VectorSubcoreMesh(core_axis_name="c", subcore_axis_name="s")

  @functools.partial(
      pl.kernel,
      out_type=jax.ShapeDtypeStruct((B_SC, D), jnp.float32),
      mesh=mesh,
      scratch_types=[
          pltpu.VMEM((CB * P, D), jnp.float32),
          pltpu.VMEM((CB * P, D), jnp.float32),
          pltpu.VMEM((CB * P, D), jnp.float32),
          pltpu.VMEM((EPW, D), jnp.float32),
          pltpu.SemaphoreType.DMA,
          pltpu.SemaphoreType.DMA,
          pltpu.SemaphoreType.DMA,
      ],
  )
  def k(x_hbm, out_hbm, buf0, buf1, buf2, stage, sem0, sem1, sem2):
    wid = lax.axis_index("s") * NC + lax.axis_index("c")
    elem0 = wid * EPW
    bufs = (buf0, buf1, buf2)
    sems = (sem0, sem1, sem2)

    def fetch(c, par):
      pltpu.async_copy(
          x_hbm.at[pl.ds((elem0 + c * CB) * P, CB * P)], bufs[par], sems[par]
      )

    def wait(c, par):
      pltpu.make_async_copy(
          x_hbm.at[pl.ds((elem0 + c * CB) * P, CB * P)], bufs[par], sems[par]
      ).wait()

    def reduce_chunk(c, par):
      buf = bufs[par]

      def elem_body(e, carry):
        base = e * P
        dsls = [pl.ds(d * LANES, LANES) for d in range(DBLK)]
        row_out = c * CB + e
        for d0 in range(0, DBLK, 4):
          grp = dsls[d0:d0 + 4]
          accs = [buf[base, dsl] for dsl in grp]
          for p in range(1, P):
            for i, dsl in enumerate(grp):
              accs[i] = accs[i] + buf[base + p, dsl]
          for i, dsl in enumerate(grp):
            stage[row_out, dsl] = accs[i]
        return carry

      lax.fori_loop(0, CB, elem_body, 0)

    fetch(0, 0)
    fetch(1, 1)
    fetch(2, 2)

    assert NCHUNKS % 3 == 0

    @pl.loop(0, NCHUNKS, step=3)
    def _ring(g):
      for j in range(3):
        wait(g + j, j)
        reduce_chunk(g + j, j)
        pl.when(g + j + 3 < NCHUNKS)(
            functools.partial(lambda jj: fetch(g + jj + 3, jj), j))

    pltpu.sync_copy(stage, out_hbm.at[pl.ds(elem0, EPW)])

  return k(x_flat)


def _mm_bias(s, w_ref, b_ref):
  y = lax.dot_general(
      s, w_ref[...], (((1,), (1,)), ((), ())),
      preferred_element_type=jnp.float32,
  )
  return y + jnp.float32(P) * b_ref[...]


def _tc_fused(x_full, w, b2d):
  def body(x_ref, w_ref, b_ref, o_ref):
    o_ref[...] = _mm_bias(jnp.sum(x_ref[...], axis=1), w_ref, b_ref)

  off = B_SC // TC_BLK
  return pl.pallas_call(
      body,
      grid=(B_TC // TC_BLK,),
      in_specs=[
          pl.BlockSpec((TC_BLK, P, D), lambda i: (i + off, 0, 0)),
          pl.BlockSpec((OUT, D), lambda i: (0, 0)),
          pl.BlockSpec((1, OUT), lambda i: (0, 0)),
      ],
      out_specs=pl.BlockSpec((TC_BLK, OUT), lambda i: (i + off, 0)),
      out_shape=jax.ShapeDtypeStruct((B, OUT), jnp.float32),
  )(x_full, w, b2d)


def _tc_linear(s, w, b2d, out_partial):
  def mm(s_ref, w_ref, b_ref, _, o_ref):
    o_ref[...] = _mm_bias(s_ref[...], w_ref, b_ref)

  return pl.pallas_call(
      mm,
      grid=(1,),
      in_specs=[
          pl.BlockSpec((B_SC, D), lambda i: (0, 0)),
          pl.BlockSpec((OUT, D), lambda i: (0, 0)),
          pl.BlockSpec((1, OUT), lambda i: (0, 0)),
          pl.BlockSpec(memory_space=pl.ANY),
      ],
      out_specs=pl.BlockSpec((B_SC, OUT), lambda i: (0, 0)),
      out_shape=jax.ShapeDtypeStruct((B, OUT), jnp.float32),
      input_output_aliases={3: 0},
  )(s, w, b2d, out_partial)


@jax.jit
def kernel(pert_batch, W, b):
  b2d = b.reshape(1, OUT)
  x_flat = pert_batch.reshape(B * P, D)
  s_sc = _sc_segment_sum(x_flat)
  out_partial = _tc_fused(pert_batch, W, b2d)
  return _tc_linear(s_sc, W, b2d, out_partial)


# speedup vs baseline: 1.0406x; 1.0318x over previous
import functools

import jax
import jax.numpy as jnp
from jax import lax
from jax.experimental import pallas as pl
from jax.experimental.pallas import tpu as pltpu
from jax.experimental.pallas import tpu_sc as plsc

B, P, D, OUT = 4096, 32, 128, 128
LANES = 16
NC, NS = 2, 16
NW = NC * NS
CB = 8
DBLK = D // LANES

B_SC = 1024
B_TC = B - B_SC
EPW = B_SC // NW
NCHUNKS = EPW // CB
TC_BLK = 512


def _sc_segment_sum(x_flat):
  mesh = plsc.VectorSubcoreMesh(core_axis_name="c", subcore_axis_name="s")

  @functools.partial(
      pl.kernel,
      out_type=jax.ShapeDtypeStruct((B_SC, D), jnp.float32),
      mesh=mesh,
      scratch_types=[
          pltpu.VMEM((CB * P, D), jnp.float32),
          pltpu.VMEM((CB * P, D), jnp.float32),
          pltpu.VMEM((CB * P, D), jnp.float32),
          pltpu.VMEM((EPW, D), jnp.float32),
          pltpu.SemaphoreType.DMA,
          pltpu.SemaphoreType.DMA,
          pltpu.SemaphoreType.DMA,
      ],
  )
  def k(x_hbm, out_hbm, buf0, buf1, buf2, stage, sem0, sem1, sem2):
    wid = lax.axis_index("s") * NC + lax.axis_index("c")
    elem0 = wid * EPW
    bufs = (buf0, buf1, buf2)
    sems = (sem0, sem1, sem2)

    def fetch(c, par):
      pltpu.async_copy(
          x_hbm.at[pl.ds((elem0 + c * CB) * P, CB * P)], bufs[par], sems[par]
      )

    def wait(c, par):
      pltpu.make_async_copy(
          x_hbm.at[pl.ds((elem0 + c * CB) * P, CB * P)], bufs[par], sems[par]
      ).wait()

    def reduce_chunk(c, par):
      buf = bufs[par]

      def elem_body(e, carry):
        base = e * P
        dsls = [pl.ds(d * LANES, LANES) for d in range(DBLK)]
        row_out = c * CB + e
        for d0 in range(0, DBLK, 4):
          grp = dsls[d0:d0 + 4]
          accs = [buf[base, dsl] for dsl in grp]
          for p in range(1, P):
            for i, dsl in enumerate(grp):
              accs[i] = accs[i] + buf[base + p, dsl]
          for i, dsl in enumerate(grp):
            stage[row_out, dsl] = accs[i]
        return carry

      lax.fori_loop(0, CB, elem_body, 0)

    DEPTH = 3 if NCHUNKS % 3 == 0 else 2
    assert NCHUNKS % DEPTH == 0
    for j in range(DEPTH):
      fetch(j, j)

    @pl.loop(0, NCHUNKS, step=DEPTH)
    def _ring(g):
      for j in range(DEPTH):
        wait(g + j, j)
        reduce_chunk(g + j, j)
        pl.when(g + j + DEPTH < NCHUNKS)(
            functools.partial(lambda jj: fetch(g + jj + DEPTH, jj), j))

    pltpu.sync_copy(stage, out_hbm.at[pl.ds(elem0, EPW)])

  return k(x_flat)


def _mm_bias(s, w_ref, b_ref):
  y = lax.dot_general(
      s, w_ref[...], (((1,), (1,)), ((), ())),
      preferred_element_type=jnp.float32,
  )
  return y + jnp.float32(P) * b_ref[...]


def _tc_fused(x_full, w, b2d):
  def body(x_ref, w_ref, b_ref, o_ref):
    o_ref[...] = _mm_bias(jnp.sum(x_ref[...], axis=1), w_ref, b_ref)

  off = B_SC // TC_BLK
  return pl.pallas_call(
      body,
      grid=(B_TC // TC_BLK,),
      in_specs=[
          pl.BlockSpec((TC_BLK, P, D), lambda i: (i + off, 0, 0)),
          pl.BlockSpec((OUT, D), lambda i: (0, 0)),
          pl.BlockSpec((1, OUT), lambda i: (0, 0)),
      ],
      out_specs=pl.BlockSpec((TC_BLK, OUT), lambda i: (i + off, 0)),
      out_shape=jax.ShapeDtypeStruct((B, OUT), jnp.float32),
  )(x_full, w, b2d)


def _tc_linear(s, w, b2d, out_partial):
  def mm(s_ref, w_ref, b_ref, _, o_ref):
    o_ref[...] = _mm_bias(s_ref[...], w_ref, b_ref)

  return pl.pallas_call(
      mm,
      grid=(1,),
      in_specs=[
          pl.BlockSpec((B_SC, D), lambda i: (0, 0)),
          pl.BlockSpec((OUT, D), lambda i: (0, 0)),
          pl.BlockSpec((1, OUT), lambda i: (0, 0)),
          pl.BlockSpec(memory_space=pl.ANY),
      ],
      out_specs=pl.BlockSpec((B_SC, OUT), lambda i: (0, 0)),
      out_shape=jax.ShapeDtypeStruct((B, OUT), jnp.float32),
      input_output_aliases={3: 0},
  )(s, w, b2d, out_partial)


@jax.jit
def kernel(pert_batch, W, b):
  b2d = b.reshape(1, OUT)
  x_flat = pert_batch.reshape(B * P, D)
  s_sc = _sc_segment_sum(x_flat)
  out_partial = _tc_fused(pert_batch, W, b2d)
  return _tc_linear(s_sc, W, b2d, out_partial)


# speedup vs baseline: 1.0738x; 1.0318x over previous
import functools

import jax
import jax.numpy as jnp
from jax import lax
from jax.experimental import pallas as pl
from jax.experimental.pallas import tpu as pltpu
from jax.experimental.pallas import tpu_sc as plsc

B, P, D, OUT = 4096, 32, 128, 128
LANES = 16
NC, NS = 2, 16
NW = NC * NS
CB = 8
DBLK = D // LANES

B_SC = 768
B_TC = B - B_SC
EPW = B_SC // NW
NCHUNKS = EPW // CB
TC_BLK = 512


def _sc_segment_sum(x_flat):
  mesh = plsc.VectorSubcoreMesh(core_axis_name="c", subcore_axis_name="s")

  @functools.partial(
      pl.kernel,
      out_type=jax.ShapeDtypeStruct((B_SC, D), jnp.float32),
      mesh=mesh,
      scratch_types=[
          pltpu.VMEM((CB * P, D), jnp.float32),
          pltpu.VMEM((CB * P, D), jnp.float32),
          pltpu.VMEM((CB * P, D), jnp.float32),
          pltpu.VMEM((EPW, D), jnp.float32),
          pltpu.SemaphoreType.DMA,
          pltpu.SemaphoreType.DMA,
          pltpu.SemaphoreType.DMA,
      ],
  )
  def k(x_hbm, out_hbm, buf0, buf1, buf2, stage, sem0, sem1, sem2):
    wid = lax.axis_index("s") * NC + lax.axis_index("c")
    elem0 = wid * EPW
    bufs = (buf0, buf1, buf2)
    sems = (sem0, sem1, sem2)

    def fetch(c, par):
      pltpu.async_copy(
          x_hbm.at[pl.ds((elem0 + c * CB) * P, CB * P)], bufs[par], sems[par]
      )

    def wait(c, par):
      pltpu.make_async_copy(
          x_hbm.at[pl.ds((elem0 + c * CB) * P, CB * P)], bufs[par], sems[par]
      ).wait()

    def reduce_chunk(c, par):
      buf = bufs[par]

      def elem_body(e, carry):
        base = e * P
        dsls = [pl.ds(d * LANES, LANES) for d in range(DBLK)]
        row_out = c * CB + e
        for d0 in range(0, DBLK, 4):
          grp = dsls[d0:d0 + 4]
          accs = [buf[base, dsl] for dsl in grp]
          for p in range(1, P):
            for i, dsl in enumerate(grp):
              accs[i] = accs[i] + buf[base + p, dsl]
          for i, dsl in enumerate(grp):
            stage[row_out, dsl] = accs[i]
        return carry

      lax.fori_loop(0, CB, elem_body, 0)

    DEPTH = 3 if NCHUNKS % 3 == 0 else 2
    assert NCHUNKS % DEPTH == 0
    for j in range(DEPTH):
      fetch(j, j)

    @pl.loop(0, NCHUNKS, step=DEPTH)
    def _ring(g):
      for j in range(DEPTH):
        wait(g + j, j)
        reduce_chunk(g + j, j)
        pl.when(g + j + DEPTH < NCHUNKS)(
            functools.partial(lambda jj: fetch(g + jj + DEPTH, jj), j))

    pltpu.sync_copy(stage, out_hbm.at[pl.ds(elem0, EPW)])

  return k(x_flat)


def _mm_bias(s, w_ref, b_ref):
  y = lax.dot_general(
      s, w_ref[...], (((1,), (1,)), ((), ())),
      preferred_element_type=jnp.float32,
  )
  return y + jnp.float32(P) * b_ref[...]


def _tc_fused(x_full, w, b2d):
  def body(x_ref, w_ref, b_ref, o_ref):
    o_ref[...] = _mm_bias(jnp.sum(x_ref[...], axis=1), w_ref, b_ref)

  off = B_SC // TC_BLK
  return pl.pallas_call(
      body,
      grid=(B_TC // TC_BLK,),
      in_specs=[
          pl.BlockSpec((TC_BLK, P, D), lambda i: (i + off, 0, 0)),
          pl.BlockSpec((OUT, D), lambda i: (0, 0)),
          pl.BlockSpec((1, OUT), lambda i: (0, 0)),
      ],
      out_specs=pl.BlockSpec((TC_BLK, OUT), lambda i: (i + off, 0)),
      out_shape=jax.ShapeDtypeStruct((B, OUT), jnp.float32),
  )(x_full, w, b2d)


def _tc_linear(s, w, b2d, out_partial):
  def mm(s_ref, w_ref, b_ref, _, o_ref):
    o_ref[...] = _mm_bias(s_ref[...], w_ref, b_ref)

  return pl.pallas_call(
      mm,
      grid=(1,),
      in_specs=[
          pl.BlockSpec((B_SC, D), lambda i: (0, 0)),
          pl.BlockSpec((OUT, D), lambda i: (0, 0)),
          pl.BlockSpec((1, OUT), lambda i: (0, 0)),
          pl.BlockSpec(memory_space=pl.ANY),
      ],
      out_specs=pl.BlockSpec((B_SC, OUT), lambda i: (0, 0)),
      out_shape=jax.ShapeDtypeStruct((B, OUT), jnp.float32),
      input_output_aliases={3: 0},
  )(s, w, b2d, out_partial)


@jax.jit
def kernel(pert_batch, W, b):
  b2d = b.reshape(1, OUT)
  x_flat = pert_batch.reshape(B * P, D)
  s_sc = _sc_segment_sum(x_flat)
  out_partial = _tc_fused(pert_batch, W, b2d)
  return _tc_linear(s_sc, W, b2d, out_partial)
